# baseline (speedup 1.0000x reference)
import functools

import numpy as np
import jax
import jax.numpy as jnp
from jax import lax
from jax.experimental import pallas as pl
from jax.experimental.pallas import tpu as pltpu
from jax.experimental.pallas import tpu_sc as plsc

D_MODEL = 768
VOCAB = 100000
BATCH = 4
SEQ = 8192

SCALE = float(np.sqrt(np.float32(D_MODEL)))

NUM_CORES = 2
NUM_SUBCORES = 16
NUM_WORKERS = NUM_CORES * NUM_SUBCORES
POS_PER_WORKER = SEQ // NUM_WORKERS
CHUNK = 32
N_CHUNKS = POS_PER_WORKER // CHUNK
LANES = 16
D_GROUPS = D_MODEL // LANES
STEP_ELEMS = CHUNK * D_MODEL


def _sinusoidal_pe(length, d_model):
    pos = np.arange(length)[:, None].astype(np.float32)
    i = np.arange(d_model)[None, :].astype(np.float32)
    angle_rates = 1.0 / np.power(10000.0, (2.0 * (i // 2)) / np.float32(d_model))
    angles = pos * angle_rates
    pe = np.zeros((length, d_model), dtype=np.float32)
    pe[:, 0::2] = np.sin(angles[:, 0::2])
    pe[:, 1::2] = np.cos(angles[:, 1::2])
    return pe


_PE_FLAT = _sinusoidal_pe(SEQ, D_MODEL).reshape(-1)

_MESH = plsc.VectorSubcoreMesh(core_axis_name="c", subcore_axis_name="s")


@functools.partial(
    pl.kernel,
    out_type=jax.ShapeDtypeStruct((BATCH * SEQ * D_MODEL,), jnp.float32),
    mesh=_MESH,
    scratch_types=[
        pltpu.VMEM((BATCH, POS_PER_WORKER), jnp.int32),
        pltpu.VMEM((CHUNK, D_MODEL), jnp.float32),
        pltpu.VMEM((CHUNK, D_MODEL), jnp.float32),
        pltpu.VMEM((STEP_ELEMS,), jnp.float32),
        pltpu.VMEM((STEP_ELEMS,), jnp.float32),
        pltpu.SemaphoreType.DMA,
        pltpu.SemaphoreType.DMA,
        pltpu.SemaphoreType.DMA,
        pltpu.SemaphoreType.DMA,
        pltpu.SemaphoreType.DMA,
        pltpu.SemaphoreType.DMA,
    ],
)
def _emb_pe_kernel(x_hbm, table_hbm, pe_hbm, out_hbm,
                   idx_v, g0, g1, o0, o1,
                   gsem0, gsem1, fsem0, fsem1, ssem0, ssem1):
    gbuf = (g0, g1)
    obuf = (o0, o1)
    gsem = (gsem0, gsem1)
    fsem = (fsem0, fsem1)
    ssem = (ssem0, ssem1)

    wid = lax.axis_index("s") * NUM_CORES + lax.axis_index("c")
    pos0 = wid * POS_PER_WORKER

    def pe_src(ci):
        return pe_hbm.at[pl.ds((pos0 + ci * CHUNK) * D_MODEL, STEP_ELEMS)]

    def gather_src(ci, b):
        return table_hbm.at[idx_v.at[b, pl.ds(ci * CHUNK, CHUNK)]]

    def out_dst(ci, b):
        return out_hbm.at[pl.ds((b * SEQ + pos0 + ci * CHUNK) * D_MODEL,
                                STEP_ELEMS)]

    def issue_f(ci, b):
        pltpu.make_async_copy(pe_src(ci), obuf[b % 2], fsem[b % 2]).start()

    def wait_f(ci, b):
        pltpu.make_async_copy(pe_src(ci), obuf[b % 2], fsem[b % 2]).wait()

    def issue_g(ci, b):
        pltpu.make_async_copy(gather_src(ci, b), gbuf[b % 2], gsem[b % 2]).start()

    def wait_g(ci, b):
        pltpu.make_async_copy(gather_src(ci, b), gbuf[b % 2], gsem[b % 2]).wait()

    def issue_s(ci, b):
        pltpu.make_async_copy(obuf[b % 2], out_dst(ci, b), ssem[b % 2]).start()

    def wait_s(ci, b):
        pltpu.make_async_copy(obuf[b % 2], out_dst(ci, b), ssem[b % 2]).wait()

    def compute(b):
        g = gbuf[b % 2]
        o = obuf[b % 2]

        @plsc.parallel_loop(0, CHUNK, unroll=2)
        def _(r):
            base = r * D_MODEL
            for gi in range(D_GROUPS):
                plsc.addupdate(o.at[pl.ds(base + gi * LANES, LANES)],
                               g[r, pl.ds(gi * LANES, LANES)] * SCALE)

    for b in range(BATCH):
        pltpu.sync_copy(x_hbm.at[b, pl.ds(pos0, POS_PER_WORKER)],
                        idx_v.at[b])

    issue_g(0, 0)
    issue_g(0, 1)
    issue_f(0, 0)

    def chunk_body(ci, carry):
        for b in range(BATCH):
            if b == 0:
                @pl.when(ci > 0)
                def _():
                    wait_s(ci - 1, BATCH - 1)
            else:
                wait_s(ci, b - 1)
            if b < BATCH - 1:
                issue_f(ci, b + 1)
            else:
                @pl.when(ci < N_CHUNKS - 1)
                def _():
                    issue_f(ci + 1, 0)
            wait_g(ci, b)
            wait_f(ci, b)
            compute(b)
            issue_s(ci, b)
            if b < 2:
                issue_g(ci, b + 2)
            else:
                @pl.when(ci < N_CHUNKS - 1)
                def _():
                    issue_g(ci + 1, b - 2)
        return carry

    lax.fori_loop(0, N_CHUNKS, chunk_body, 0)

    wait_s(N_CHUNKS - 1, BATCH - 1)


def kernel(x, table):
    pe = jnp.asarray(_PE_FLAT)
    out_flat = _emb_pe_kernel(x.astype(jnp.int32), table, pe)
    return out_flat.reshape(BATCH, SEQ, D_MODEL)


# speedup vs baseline: 1.7149x; 1.7149x over previous
import functools

import numpy as np
import jax
import jax.numpy as jnp
from jax import lax
from jax.experimental import pallas as pl
from jax.experimental.pallas import tpu as pltpu
from jax.experimental.pallas import tpu_sc as plsc

D_MODEL = 768
VOCAB = 100000
BATCH = 4
SEQ = 8192

SCALE = float(np.sqrt(np.float32(D_MODEL)))

NUM_CORES = 2
NUM_SUBCORES = 16
NUM_WORKERS = NUM_CORES * NUM_SUBCORES
POS_PER_WORKER = SEQ // NUM_WORKERS
CHUNK = 32
N_CHUNKS = POS_PER_WORKER // CHUNK
LANES = 16
D_GROUPS = D_MODEL // LANES
STEP_ELEMS = CHUNK * D_MODEL


def _sinusoidal_pe(length, d_model):
    pos = np.arange(length)[:, None].astype(np.float32)
    i = np.arange(d_model)[None, :].astype(np.float32)
    angle_rates = 1.0 / np.power(10000.0, (2.0 * (i // 2)) / np.float32(d_model))
    angles = pos * angle_rates
    pe = np.zeros((length, d_model), dtype=np.float32)
    pe[:, 0::2] = np.sin(angles[:, 0::2])
    pe[:, 1::2] = np.cos(angles[:, 1::2])
    return pe


_PE = _sinusoidal_pe(SEQ, D_MODEL)

_MESH = plsc.VectorSubcoreMesh(core_axis_name="c", subcore_axis_name="s")


@functools.partial(
    pl.kernel,
    out_type=jax.ShapeDtypeStruct((BATCH, SEQ, D_MODEL), jnp.float32),
    mesh=_MESH,
    scratch_types=[
        pltpu.VMEM((BATCH, POS_PER_WORKER), jnp.int32),
        pltpu.VMEM((CHUNK, D_MODEL), jnp.float32),
        pltpu.VMEM((CHUNK, D_MODEL), jnp.float32),
        pltpu.VMEM((CHUNK, D_MODEL), jnp.float32),
        pltpu.VMEM((CHUNK, D_MODEL), jnp.float32),
        pltpu.SemaphoreType.DMA,
        pltpu.SemaphoreType.DMA,
        pltpu.SemaphoreType.DMA,
        pltpu.SemaphoreType.DMA,
        pltpu.SemaphoreType.DMA,
        pltpu.SemaphoreType.DMA,
    ],
)
def _emb_pe_kernel(x_hbm, table_hbm, pe_hbm, out_hbm,
                   idx_v, g0, g1, o0, o1,
                   gsem0, gsem1, fsem0, fsem1, ssem0, ssem1):
    gbuf = (g0, g1)
    obuf = (o0, o1)
    gsem = (gsem0, gsem1)
    fsem = (fsem0, fsem1)
    ssem = (ssem0, ssem1)

    wid = lax.axis_index("s") * NUM_CORES + lax.axis_index("c")
    pos0 = wid * POS_PER_WORKER

    def pe_src(ci):
        return pe_hbm.at[pl.ds(pos0 + ci * CHUNK, CHUNK)]

    def gather_src(ci, b):
        return table_hbm.at[idx_v.at[b, pl.ds(ci * CHUNK, CHUNK)]]

    def out_dst(ci, b):
        return out_hbm.at[b, pl.ds(pos0 + ci * CHUNK, CHUNK)]

    def issue_f(ci, b):
        pltpu.make_async_copy(pe_src(ci), obuf[b % 2], fsem[b % 2]).start()

    def wait_f(ci, b):
        pltpu.make_async_copy(pe_src(ci), obuf[b % 2], fsem[b % 2]).wait()

    def issue_g(ci, b):
        pltpu.make_async_copy(gather_src(ci, b), gbuf[b % 2], gsem[b % 2]).start()

    def wait_g(ci, b):
        pltpu.make_async_copy(gather_src(ci, b), gbuf[b % 2], gsem[b % 2]).wait()

    def issue_s(ci, b):
        pltpu.make_async_copy(obuf[b % 2], out_dst(ci, b), ssem[b % 2]).start()

    def wait_s(ci, b):
        pltpu.make_async_copy(obuf[b % 2], out_dst(ci, b), ssem[b % 2]).wait()

    def compute(b):
        g = gbuf[b % 2]
        o = obuf[b % 2]

        @plsc.parallel_loop(0, CHUNK, unroll=2)
        def _(r):
            for gi in range(D_GROUPS):
                sl = pl.ds(gi * LANES, LANES)
                plsc.addupdate(o.at[r, sl], g[r, sl] * SCALE)

    for b in range(BATCH):
        pltpu.sync_copy(x_hbm.at[b, pl.ds(pos0, POS_PER_WORKER)],
                        idx_v.at[b])

    issue_g(0, 0)
    issue_g(0, 1)
    issue_f(0, 0)

    def chunk_body(ci, carry):
        for b in range(BATCH):
            if b == 0:
                @pl.when(ci > 0)
                def _():
                    wait_s(ci - 1, BATCH - 1)
            else:
                wait_s(ci, b - 1)
            if b < BATCH - 1:
                issue_f(ci, b + 1)
            else:
                @pl.when(ci < N_CHUNKS - 1)
                def _():
                    issue_f(ci + 1, 0)
            wait_g(ci, b)
            wait_f(ci, b)
            compute(b)
            issue_s(ci, b)
            if b < 2:
                issue_g(ci, b + 2)
            else:
                @pl.when(ci < N_CHUNKS - 1)
                def _():
                    issue_g(ci + 1, b - 2)
        return carry

    lax.fori_loop(0, N_CHUNKS, chunk_body, 0)

    wait_s(N_CHUNKS - 1, BATCH - 1)


def kernel(x, table):
    pe = jnp.asarray(_PE)
    return _emb_pe_kernel(x.astype(jnp.int32), table, pe)


# speedup vs baseline: 1.7372x; 1.0130x over previous
import functools

import numpy as np
import jax
import jax.numpy as jnp
from jax import lax
from jax.experimental import pallas as pl
from jax.experimental.pallas import tpu as pltpu
from jax.experimental.pallas import tpu_sc as plsc

D_MODEL = 768
VOCAB = 100000
BATCH = 4
SEQ = 8192

SCALE = float(np.sqrt(np.float32(D_MODEL)))

NUM_CORES = 2
NUM_SUBCORES = 16
NUM_WORKERS = NUM_CORES * NUM_SUBCORES
POS_PER_WORKER = SEQ // NUM_WORKERS
CHUNK = 32
N_CHUNKS = POS_PER_WORKER // CHUNK
LANES = 16
D_GROUPS = D_MODEL // LANES
STEP_ELEMS = CHUNK * D_MODEL


def _sinusoidal_pe(length, d_model):
    pos = np.arange(length)[:, None].astype(np.float32)
    i = np.arange(d_model)[None, :].astype(np.float32)
    angle_rates = 1.0 / np.power(10000.0, (2.0 * (i // 2)) / np.float32(d_model))
    angles = pos * angle_rates
    pe = np.zeros((length, d_model), dtype=np.float32)
    pe[:, 0::2] = np.sin(angles[:, 0::2])
    pe[:, 1::2] = np.cos(angles[:, 1::2])
    return pe


_PE = _sinusoidal_pe(SEQ, D_MODEL)

_MESH = plsc.VectorSubcoreMesh(core_axis_name="c", subcore_axis_name="s")


@functools.partial(
    pl.kernel,
    out_type=jax.ShapeDtypeStruct((BATCH, SEQ, D_MODEL), jnp.float32),
    mesh=_MESH,
    scratch_types=[
        pltpu.VMEM((BATCH, POS_PER_WORKER), jnp.int32),
        pltpu.VMEM((CHUNK, D_MODEL), jnp.float32),
        pltpu.VMEM((CHUNK, D_MODEL), jnp.float32),
        pltpu.VMEM((CHUNK, D_MODEL), jnp.float32),
        pltpu.VMEM((CHUNK, D_MODEL), jnp.float32),
        pltpu.SemaphoreType.DMA,
        pltpu.SemaphoreType.DMA,
        pltpu.SemaphoreType.DMA,
        pltpu.SemaphoreType.DMA,
        pltpu.SemaphoreType.DMA,
        pltpu.SemaphoreType.DMA,
    ],
)
def _emb_pe_kernel(x_hbm, table_hbm, pe_hbm, out_hbm,
                   idx_v, g0, g1, o0, o1,
                   gsem0, gsem1, fsem0, fsem1, ssem0, ssem1):
    gbuf = (g0, g1)
    obuf = (o0, o1)
    gsem = (gsem0, gsem1)
    fsem = (fsem0, fsem1)
    ssem = (ssem0, ssem1)

    wid = lax.axis_index("s") * NUM_CORES + lax.axis_index("c")
    pos0 = wid * POS_PER_WORKER

    def pe_src(ci):
        return pe_hbm.at[pl.ds(pos0 + ci * CHUNK, CHUNK)]

    def gather_src(ci, b):
        return table_hbm.at[idx_v.at[b, pl.ds(ci * CHUNK, CHUNK)]]

    def out_dst(ci, b):
        return out_hbm.at[b, pl.ds(pos0 + ci * CHUNK, CHUNK)]

    def issue_f(ci, b):
        pltpu.make_async_copy(pe_src(ci), obuf[b % 2], fsem[b % 2]).start()

    def wait_f(ci, b):
        pltpu.make_async_copy(pe_src(ci), obuf[b % 2], fsem[b % 2]).wait()

    def issue_g(ci, b):
        pltpu.make_async_copy(gather_src(ci, b), gbuf[b % 2], gsem[b % 2]).start()

    def wait_g(ci, b):
        pltpu.make_async_copy(gather_src(ci, b), gbuf[b % 2], gsem[b % 2]).wait()

    def issue_s(ci, b):
        pltpu.make_async_copy(obuf[b % 2], out_dst(ci, b), ssem[b % 2]).start()

    def wait_s(ci, b):
        pltpu.make_async_copy(obuf[b % 2], out_dst(ci, b), ssem[b % 2]).wait()

    def compute(b):
        g = gbuf[b % 2]
        o = obuf[b % 2]

        @plsc.parallel_loop(0, CHUNK)
        def _(r):
            @plsc.parallel_loop(0, D_MODEL, step=LANES, unroll=8)
            def _(c):
                sl = pl.ds(c, LANES)
                plsc.addupdate(o.at[r, sl], g[r, sl] * SCALE)

    for b in range(BATCH):
        pltpu.sync_copy(x_hbm.at[b, pl.ds(pos0, POS_PER_WORKER)],
                        idx_v.at[b])

    issue_g(0, 0)
    issue_g(0, 1)
    issue_f(0, 0)

    def chunk_body(ci, carry):
        for b in range(BATCH):
            if b == 0:
                @pl.when(ci > 0)
                def _():
                    wait_s(ci - 1, BATCH - 1)
            else:
                wait_s(ci, b - 1)
            if b < BATCH - 1:
                issue_f(ci, b + 1)
            else:
                @pl.when(ci < N_CHUNKS - 1)
                def _():
                    issue_f(ci + 1, 0)
            wait_g(ci, b)
            wait_f(ci, b)
            compute(b)
            issue_s(ci, b)
            if b < 2:
                issue_g(ci, b + 2)
            else:
                @pl.when(ci < N_CHUNKS - 1)
                def _():
                    issue_g(ci + 1, b - 2)
        return carry

    lax.fori_loop(0, N_CHUNKS, chunk_body, 0)

    wait_s(N_CHUNKS - 1, BATCH - 1)


def kernel(x, table):
    pe = jnp.asarray(_PE)
    return _emb_pe_kernel(x.astype(jnp.int32), table, pe)
